# baseline (speedup 1.0000x reference)
import functools

import jax
import jax.numpy as jnp
from jax import lax
from jax.experimental import pallas as pl
from jax.experimental.pallas import tpu as pltpu
from jax.experimental.pallas import tpu_sc as plsc

D = 64
SCALE = 8.0
L = 16
NC = 2
NS = 16
NW = NC * NS
CHUNK = 512


@functools.partial(jax.jit, static_argnames=("n_idx",))
def _embed(x_flat, table, n_idx):
    b_per_w = n_idx // NW
    n_chunks = b_per_w // CHUNK
    mesh = plsc.VectorSubcoreMesh(core_axis_name="c", subcore_axis_name="s")

    @functools.partial(
        pl.kernel,
        out_type=jax.ShapeDtypeStruct((n_idx, D), jnp.float32),
        mesh=mesh,
        compiler_params=pltpu.CompilerParams(use_tc_tiling_on_sc=False),
        scratch_types=[
            pltpu.VMEM((CHUNK,), jnp.int32),
            pltpu.VMEM((CHUNK, D), jnp.float32),
            pltpu.SemaphoreType.DMA,
        ],
    )
    def k(x_hbm, table_hbm, out_hbm, idx_v, rows_v, sem):
        wid = lax.axis_index("s") * NC + lax.axis_index("c")
        base = wid * b_per_w

        def chunk_body(c, carry):
            off = base + c * CHUNK
            pltpu.sync_copy(x_hbm.at[pl.ds(off, CHUNK)], idx_v)
            pltpu.async_copy(table_hbm.at[idx_v], rows_v, sem).wait()

            def scale_body(r, carry2):
                for j in range(D // L):
                    sl = pl.ds(j * L, L)
                    rows_v[r, sl] = rows_v[r, sl] * SCALE
                return carry2

            lax.fori_loop(0, CHUNK, scale_body, 0, unroll=4)
            pltpu.sync_copy(rows_v, out_hbm.at[pl.ds(off, CHUNK)])
            return carry

        lax.fori_loop(0, n_chunks, chunk_body, 0)

    return k(x_flat, table)


def kernel(x, table):
    b, s = x.shape
    x_flat = x.reshape(-1)
    out = _embed(x_flat, table, x_flat.shape[0])
    return out.reshape(b, s, D)


# speedup vs baseline: 1.1511x; 1.1511x over previous
import functools

import jax
import jax.numpy as jnp
from jax import lax
from jax.experimental import pallas as pl
from jax.experimental.pallas import tpu as pltpu
from jax.experimental.pallas import tpu_sc as plsc

D = 64
SCALE = 8.0
L = 16
NC = 2
NS = 16
NW = NC * NS
V = 1000000
CHUNK = 512


@functools.partial(jax.jit, static_argnames=("n_idx",))
def _embed(x2_flat, tab2, n_idx):
    b_per_w = n_idx // NW
    n_chunks = b_per_w // CHUNK
    mesh = plsc.VectorSubcoreMesh(core_axis_name="c", subcore_axis_name="s")

    @functools.partial(
        pl.kernel,
        out_type=jax.ShapeDtypeStruct((n_idx, 128), jnp.float32),
        mesh=mesh,
        scratch_types=[
            pltpu.VMEM((CHUNK,), jnp.int32),
            pltpu.VMEM((CHUNK, 128), jnp.float32),
            pltpu.SemaphoreType.DMA,
        ],
    )
    def k(x_hbm, tab_hbm, out_hbm, idx_v, rows_v, sem):
        wid = lax.axis_index("s") * NC + lax.axis_index("c")
        base = wid * b_per_w

        def chunk_body(c, carry):
            off = base + c * CHUNK
            pltpu.sync_copy(x_hbm.at[pl.ds(off, CHUNK)], idx_v)
            pltpu.async_copy(tab_hbm.at[idx_v], rows_v, sem).wait()

            def scale_body(r, carry2):
                for j in range(128 // L):
                    sl = pl.ds(j * L, L)
                    rows_v[r, sl] = rows_v[r, sl] * SCALE
                return carry2

            lax.fori_loop(0, CHUNK, scale_body, 0, unroll=4)
            pltpu.sync_copy(rows_v, out_hbm.at[pl.ds(off, CHUNK), :])
            return carry

        lax.fori_loop(0, n_chunks, chunk_body, 0)

    return k(x2_flat, tab2)


def kernel(x, table):
    b, s = x.shape
    tpad = jnp.pad(table, ((0, 0), (0, 64)))
    x_flat = x.reshape(-1)
    out_pad = _embed(x_flat, tpad, b * s)
    return out_pad[:, :D].reshape(b, s, D)


# speedup vs baseline: 1.3308x; 1.1560x over previous
import functools

import jax
import jax.numpy as jnp
from jax import lax
from jax.experimental import pallas as pl
from jax.experimental.pallas import tpu as pltpu
from jax.experimental.pallas import tpu_sc as plsc

D = 64
SCALE = 8.0
L = 16
NC = 2
NS = 16
NW = NC * NS
V = 1000000
CHUNK = 256


@functools.partial(jax.jit, static_argnames=("n_idx",))
def _embed(x_flat, tab, n_idx):
    b_per_w = n_idx // NW
    n_chunks = b_per_w // CHUNK
    mesh = plsc.VectorSubcoreMesh(core_axis_name="c", subcore_axis_name="s")

    @functools.partial(
        pl.kernel,
        out_type=jax.ShapeDtypeStruct((n_idx, 128), jnp.float32),
        mesh=mesh,
        scratch_types=[
            pltpu.VMEM((b_per_w,), jnp.int32),
            pltpu.VMEM((2, CHUNK, 128), jnp.float32),
            pltpu.SemaphoreType.DMA((2,)),
        ],
    )
    def k(x_hbm, tab_hbm, out_hbm, idx_all, rows2, gsem):
        wid = lax.axis_index("s") * NC + lax.axis_index("c")
        base = wid * b_per_w
        pltpu.sync_copy(x_hbm.at[pl.ds(base, b_per_w)], idx_all)

        def start_gather(c, buf):
            pltpu.async_copy(
                tab_hbm.at[idx_all.at[pl.ds(c * CHUNK, CHUNK)]],
                rows2.at[buf],
                gsem.at[buf],
            )

        start_gather(0, 0)

        def chunk_body(c, carry):
            cur = c % 2

            @pl.when(c + 1 < n_chunks)
            def _issue_next():
                start_gather(c + 1, (c + 1) % 2)

            pltpu.make_async_copy(
                tab_hbm.at[idx_all.at[pl.ds(c * CHUNK, CHUNK)]],
                rows2.at[cur],
                gsem.at[cur],
            ).wait()

            def scale_body(r, carry2):
                for j in range(128 // L):
                    sl = pl.ds(j * L, L)
                    rows2[cur, r, sl] = rows2[cur, r, sl] * SCALE
                return carry2

            lax.fori_loop(0, CHUNK, scale_body, 0, unroll=4)
            pltpu.sync_copy(
                rows2.at[cur], out_hbm.at[pl.ds(base + c * CHUNK, CHUNK), :]
            )
            return carry

        lax.fori_loop(0, n_chunks, chunk_body, 0)

    return k(x_flat, tab)


def kernel(x, table):
    b, s = x.shape
    n = b * s
    tpad = jnp.pad(table, ((0, 0), (0, 64)))
    x_flat = x.reshape(-1)
    out_pad = _embed(x_flat, tpad, n)
    return out_pad[:, :D].reshape(b, s, D)
